# baseline (speedup 1.0000x reference)
import functools

import jax
import jax.numpy as jnp
from jax import lax
from jax.experimental import pallas as pl
from jax.experimental.pallas import tpu as pltpu
from jax.experimental.pallas import tpu_sc as plsc

NC = 2
NS = 16
CH = 128


def _tc_table_kernel(x_ref, w_ref, bvec_ref, src_ref, et_ref, tab_ref,
                     gidx_ref, *, plane_rows, spp, bias_plane):
    acc = jnp.dot(x_ref[...], w_ref[...], preferred_element_type=jnp.float32)

    @pl.when(pl.program_id(0) == bias_plane)
    def _():
        tab_ref[0] = acc + bvec_ref[...]

    @pl.when(pl.program_id(0) != bias_plane)
    def _():
        tab_ref[0] = acc

    @pl.when(pl.program_id(0) == 0)
    def _():
        et = et_ref[...]
        gidx_ref[...] = ((et // spp) * plane_rows + src_ref[...] * spp
                         + (et % spp))


def _tc_combine_kernel(p_ref, o_ref):
    o_ref[...] = p_ref[0] + p_ref[1]


def _sc_scatter_body(table_hbm, gidx_hbm, dst_hbm, zrows_hbm, tbase_hbm,
                     out_hbm, gidx_v, dst_v, bufs, vout, acc_sh,
                     gsems, ssems, *, nchunk, rows_per_tile, nbuf, n_nodes,
                     bias_plane, bias_slot):
    c = lax.axis_index("c")
    s = lax.axis_index("s")
    wid = s * NC + c

    pltpu.sync_copy(gidx_hbm.at[wid], gidx_v)
    pltpu.sync_copy(dst_hbm.at[wid], dst_v)

    pltpu.sync_copy(zrows_hbm, vout)
    row0 = s * rows_per_tile
    tail = n_nodes - (NS - 1) * rows_per_tile

    @pl.when(c == 0)
    def _():
        @pl.when(s < NS - 1)
        def _():
            pltpu.sync_copy(
                tbase_hbm.at[bias_plane, pl.ds(row0, rows_per_tile),
                             bias_slot], vout)

        @pl.when(s == NS - 1)
        def _():
            pltpu.sync_copy(
                tbase_hbm.at[bias_plane, pl.ds((NS - 1) * rows_per_tile,
                                               tail), bias_slot],
                vout.at[pl.ds(0, tail)])

    pltpu.sync_copy(vout, acc_sh.at[pl.ds(row0, rows_per_tile)])
    plsc.subcore_barrier()

    def start_gather(j, b):
        pltpu.async_copy(table_hbm.at[gidx_v.at[j]], bufs.at[b], gsems.at[b])

    def wait_gather(b):
        pltpu.make_async_copy(table_hbm.at[gidx_v.at[0]], bufs.at[b],
                              gsems.at[b]).wait()

    def start_scatter(j, b):
        pltpu.async_copy(bufs.at[b], acc_sh.at[dst_v.at[j]], ssems.at[b],
                         add=True)

    def wait_scatter(b):
        pltpu.make_async_copy(bufs.at[b], acc_sh.at[dst_v.at[0]],
                              ssems.at[b]).wait()

    start_gather(0, 0)
    start_gather(1, 1)
    wait_gather(0)
    start_scatter(0, 0)
    start_gather(2, 2)

    def body(i, carry):
        for t in range(3):
            j = 3 * i + 1 + t
            b = (1 + t) % 3
            wait_gather(b)
            wait_scatter((b + 2) % 3)
            start_scatter(j, b)

            @pl.when(j + 2 < nchunk)
            def _(j=j, b=b):
                start_gather(j + 2, (b + 2) % 3)
        return carry

    lax.fori_loop(0, (nchunk - 1) // 3, body, 0)
    wait_scatter((nchunk - 1) % 3)
    plsc.subcore_barrier()

    pltpu.sync_copy(acc_sh.at[pl.ds(s * rows_per_tile, rows_per_tile)], vout)
    pltpu.sync_copy(vout, out_hbm.at[c, pl.ds(s * rows_per_tile, rows_per_tile)])


def kernel(x, edge_index, edge_type, W, loop_weight, bias):
    n, d_in = x.shape
    e = edge_type.shape[0]
    r = W.shape[0]
    d_out = W.shape[2]
    f32 = jnp.float32

    slot = 8
    spp = 128 // slot
    r_pad = ((r + 1 + spp - 1) // spp) * spp
    n_acc = ((n + 1 + 8 * NS - 1) // (8 * NS)) * (8 * NS)
    rows_per_tile = n_acc // NS
    dummy = n
    nw = NC * NS
    nbuf = 3
    nchunk = -(-e // (nw * CH))
    while (nchunk - 1) % 3:
        nchunk += 1
    e_pad = nw * nchunk * CH
    ep_rows = e_pad // 128

    w_full = jnp.concatenate([W, loop_weight[None]], axis=0)
    w_pad = jnp.zeros((r_pad, d_in, slot), f32).at[:r + 1, :, :d_out].set(w_full)
    w_cat = w_pad.transpose(1, 0, 2).reshape(d_in, r_pad * slot)
    bias_plane = r // spp
    bias_slot = r % spp
    bvec = jnp.zeros((1, 128), f32).at[0, bias_slot * slot:
                                       bias_slot * slot + d_out].set(
        bias.astype(f32))

    pad = e_pad - e
    src_r = jnp.pad(edge_index[0], (0, pad)).reshape(ep_rows, 128)
    et_r = jnp.pad(edge_type, (0, pad)).reshape(ep_rows, 128)
    dst_r = jnp.pad(edge_index[1], (0, pad), constant_values=dummy).reshape(
        nw, nchunk, CH)

    zrows = jnp.zeros((rows_per_tile, slot), f32)

    planes = r_pad * slot // 128
    plane_rows = n * 128 // slot
    table, gidx = pl.pallas_call(
        functools.partial(_tc_table_kernel, plane_rows=plane_rows, spp=spp,
                          bias_plane=bias_plane),
        grid=(planes,),
        in_specs=[
            pl.BlockSpec((n, d_in), lambda g: (0, 0)),
            pl.BlockSpec((d_in, 128), lambda g: (0, g)),
            pl.BlockSpec((1, 128), lambda g: (0, 0)),
            pl.BlockSpec((ep_rows, 128), lambda g: (0, 0)),
            pl.BlockSpec((ep_rows, 128), lambda g: (0, 0)),
        ],
        out_specs=[
            pl.BlockSpec((1, n, 128), lambda g: (g, 0, 0)),
            pl.BlockSpec((ep_rows, 128), lambda g: (0, 0)),
        ],
        out_shape=[
            jax.ShapeDtypeStruct((planes, n, 128), f32),
            jax.ShapeDtypeStruct((ep_rows, 128), jnp.int32),
        ],
    )(x, w_cat, bvec, src_r, et_r)

    mesh = plsc.VectorSubcoreMesh(core_axis_name="c", subcore_axis_name="s",
                                  num_cores=NC, num_subcores=NS)
    sc = pl.kernel(
        functools.partial(_sc_scatter_body, nchunk=nchunk,
                          rows_per_tile=rows_per_tile, nbuf=nbuf,
                          n_nodes=n, bias_plane=bias_plane,
                          bias_slot=bias_slot),
        out_type=jax.ShapeDtypeStruct((NC, n_acc, slot), f32),
        mesh=mesh,
        compiler_params=pltpu.CompilerParams(use_tc_tiling_on_sc=False),
        scratch_types=[
            pltpu.VMEM((nchunk, CH), jnp.int32),
            pltpu.VMEM((nchunk, CH), jnp.int32),
            pltpu.VMEM((nbuf, CH, slot), f32),
            pltpu.VMEM((rows_per_tile, slot), f32),
            pltpu.VMEM_SHARED((n_acc, slot), f32),
            pltpu.SemaphoreType.DMA((nbuf,)),
            pltpu.SemaphoreType.DMA((nbuf,)),
        ],
    )
    tbase = table.reshape(planes, plane_rows // spp, spp, slot)
    partials = sc(table.reshape(n * r_pad, slot),
                  gidx.reshape(nw, nchunk, CH), dst_r, zrows, tbase)

    flat_rows = n_acc * slot // 128
    out_flat = pl.pallas_call(
        _tc_combine_kernel,
        grid=(1,),
        in_specs=[
            pl.BlockSpec((NC, flat_rows, 128), lambda g: (0, 0, 0)),
        ],
        out_specs=pl.BlockSpec((flat_rows, 128), lambda g: (0, 0)),
        out_shape=jax.ShapeDtypeStruct((flat_rows, 128), f32),
    )(partials.reshape(NC, flat_rows, 128))

    return out_flat.reshape(n_acc, slot)[:n, :d_out]


# speedup vs baseline: 2.2954x; 2.2954x over previous
import functools

import jax
import jax.numpy as jnp
from jax import lax
from jax.experimental import pallas as pl
from jax.experimental.pallas import tpu as pltpu
from jax.experimental.pallas import tpu_sc as plsc

NC = 2
NS = 16
CH = 128


def _tc_table_kernel(x_ref, w_ref, bvec_ref, src_ref, et_ref, dst_ref,
                     tab_ref, gidx_ref, dst_out_ref, *, plane_rows, spp,
                     bias_plane, n_edges, n_nodes, dummy, n_rel):
    acc = jnp.dot(x_ref[...], w_ref[...], preferred_element_type=jnp.float32)

    @pl.when(pl.program_id(0) == bias_plane)
    def _():
        tab_ref[0] = acc + bvec_ref[...]

    @pl.when(pl.program_id(0) != bias_plane)
    def _():
        tab_ref[0] = acc

    @pl.when(pl.program_id(0) == 0)
    def _():
        shp = gidx_ref.shape
        eid = (lax.broadcasted_iota(jnp.int32, shp, 0) * 128
               + lax.broadcasted_iota(jnp.int32, shp, 1))
        node = eid - n_edges
        et = jnp.where(eid < n_edges, et_ref[...], n_rel)
        src = jnp.where(eid < n_edges, src_ref[...], node)
        is_pad = eid >= n_edges + n_nodes
        gidx = (et // spp) * plane_rows + src * spp + (et % spp)
        gidx_ref[...] = jnp.where(is_pad, 0, gidx)
        dst = jnp.where(eid < n_edges, dst_ref[...], node)
        dst_out_ref[...] = jnp.where(is_pad, dummy, dst)


def _tc_combine_kernel(p_ref, o_ref):
    o_ref[...] = p_ref[0] + p_ref[1]


def _sc_scatter_body(table_hbm, gidx_hbm, dst_hbm, zrows_hbm, out_hbm,
                     gidx_v, dst_v, bufs, vout, acc_sh,
                     gsems, ssems, *, nchunk, rows_per_tile, nbuf):
    c = lax.axis_index("c")
    s = lax.axis_index("s")
    wid = s * NC + c

    pltpu.sync_copy(gidx_hbm.at[wid], gidx_v)
    pltpu.sync_copy(dst_hbm.at[wid], dst_v)

    pltpu.sync_copy(zrows_hbm, vout)
    pltpu.sync_copy(vout, acc_sh.at[pl.ds(s * rows_per_tile, rows_per_tile)])
    plsc.subcore_barrier()

    def start_gather(j, b):
        pltpu.async_copy(table_hbm.at[gidx_v.at[j]], bufs.at[b], gsems.at[b])

    def wait_gather(b):
        pltpu.make_async_copy(table_hbm.at[gidx_v.at[0]], bufs.at[b],
                              gsems.at[b]).wait()

    def start_scatter(j, b):
        pltpu.async_copy(bufs.at[b], acc_sh.at[dst_v.at[j]], ssems.at[b],
                         add=True)

    def wait_scatter(b):
        pltpu.make_async_copy(bufs.at[b], acc_sh.at[dst_v.at[0]],
                              ssems.at[b]).wait()

    start_gather(0, 0)
    start_gather(1, 1)
    wait_gather(0)
    start_scatter(0, 0)
    start_gather(2, 2)

    def body(i, carry):
        for t in range(3):
            j = 3 * i + 1 + t
            b = (1 + t) % 3
            wait_gather(b)
            wait_scatter((b + 2) % 3)
            start_scatter(j, b)

            @pl.when(j + 2 < nchunk)
            def _(j=j, b=b):
                start_gather(j + 2, (b + 2) % 3)
        return carry

    lax.fori_loop(0, (nchunk - 1) // 3, body, 0)
    wait_scatter((nchunk - 1) % 3)
    plsc.subcore_barrier()

    pltpu.sync_copy(acc_sh.at[pl.ds(s * rows_per_tile, rows_per_tile)], vout)
    pltpu.sync_copy(vout, out_hbm.at[c, pl.ds(s * rows_per_tile, rows_per_tile)])


def kernel(x, edge_index, edge_type, W, loop_weight, bias):
    n, d_in = x.shape
    e = edge_type.shape[0]
    r = W.shape[0]
    d_out = W.shape[2]
    f32 = jnp.float32

    slot = 8
    spp = 128 // slot
    r_pad = ((r + 1 + spp - 1) // spp) * spp
    n_acc = ((n + 1 + 8 * NS - 1) // (8 * NS)) * (8 * NS)
    rows_per_tile = n_acc // NS
    dummy = n
    nw = NC * NS
    nbuf = 3
    e_full = e + n
    nchunk = -(-e_full // (nw * CH))
    while (nchunk - 1) % 3:
        nchunk += 1
    e_pad = nw * nchunk * CH
    ep_rows = e_pad // 128

    w_full = jnp.concatenate([W, loop_weight[None]], axis=0)
    w_pad = jnp.zeros((r_pad, d_in, slot), f32).at[:r + 1, :, :d_out].set(w_full)
    w_cat = w_pad.transpose(1, 0, 2).reshape(d_in, r_pad * slot)
    bias_plane = r // spp
    bias_slot = r % spp
    bvec = jnp.zeros((1, 128), f32).at[0, bias_slot * slot:
                                       bias_slot * slot + d_out].set(
        bias.astype(f32))

    e_rows = e // 128
    rpad = ((0, ep_rows - e_rows), (0, 0))
    src_r = jnp.pad(edge_index[0].reshape(e_rows, 128), rpad)
    et_r = jnp.pad(edge_type.reshape(e_rows, 128), rpad)
    dst_in = jnp.pad(edge_index[1].reshape(e_rows, 128), rpad)

    zrows = jnp.zeros((rows_per_tile, slot), f32)

    planes = r_pad * slot // 128
    plane_rows = n * 128 // slot
    table, gidx, dst_syn = pl.pallas_call(
        functools.partial(_tc_table_kernel, plane_rows=plane_rows, spp=spp,
                          bias_plane=bias_plane, n_edges=e, n_nodes=n,
                          dummy=dummy, n_rel=r),
        grid=(planes,),
        in_specs=[
            pl.BlockSpec((n, d_in), lambda g: (0, 0)),
            pl.BlockSpec((d_in, 128), lambda g: (0, g)),
            pl.BlockSpec((1, 128), lambda g: (0, 0)),
            pl.BlockSpec((ep_rows, 128), lambda g: (0, 0)),
            pl.BlockSpec((ep_rows, 128), lambda g: (0, 0)),
            pl.BlockSpec((ep_rows, 128), lambda g: (0, 0)),
        ],
        out_specs=[
            pl.BlockSpec((1, n, 128), lambda g: (g, 0, 0)),
            pl.BlockSpec((ep_rows, 128), lambda g: (0, 0)),
            pl.BlockSpec((ep_rows, 128), lambda g: (0, 0)),
        ],
        out_shape=[
            jax.ShapeDtypeStruct((planes, n, 128), f32),
            jax.ShapeDtypeStruct((ep_rows, 128), jnp.int32),
            jax.ShapeDtypeStruct((ep_rows, 128), jnp.int32),
        ],
    )(x, w_cat, bvec, src_r, et_r, dst_in)

    mesh = plsc.VectorSubcoreMesh(core_axis_name="c", subcore_axis_name="s",
                                  num_cores=NC, num_subcores=NS)
    sc = pl.kernel(
        functools.partial(_sc_scatter_body, nchunk=nchunk,
                          rows_per_tile=rows_per_tile, nbuf=nbuf),
        out_type=jax.ShapeDtypeStruct((NC, n_acc, slot), f32),
        mesh=mesh,
        compiler_params=pltpu.CompilerParams(use_tc_tiling_on_sc=False),
        scratch_types=[
            pltpu.VMEM((nchunk, CH), jnp.int32),
            pltpu.VMEM((nchunk, CH), jnp.int32),
            pltpu.VMEM((nbuf, CH, slot), f32),
            pltpu.VMEM((rows_per_tile, slot), f32),
            pltpu.VMEM_SHARED((n_acc, slot), f32),
            pltpu.SemaphoreType.DMA((nbuf,)),
            pltpu.SemaphoreType.DMA((nbuf,)),
        ],
    )
    partials = sc(table.reshape(n * r_pad, slot),
                  gidx.reshape(nw, nchunk, CH),
                  dst_syn.reshape(nw, nchunk, CH), zrows)

    flat_rows = n_acc * slot // 128
    out_flat = pl.pallas_call(
        _tc_combine_kernel,
        grid=(1,),
        in_specs=[
            pl.BlockSpec((NC, flat_rows, 128), lambda g: (0, 0, 0)),
        ],
        out_specs=pl.BlockSpec((flat_rows, 128), lambda g: (0, 0)),
        out_shape=jax.ShapeDtypeStruct((flat_rows, 128), f32),
    )(partials.reshape(NC, flat_rows, 128))

    return out_flat.reshape(n_acc, slot)[:n, :d_out]


# speedup vs baseline: 2.5421x; 1.1075x over previous
import functools

import jax
import jax.numpy as jnp
from jax import lax
from jax.experimental import pallas as pl
from jax.experimental.pallas import tpu as pltpu
from jax.experimental.pallas import tpu_sc as plsc

NC = 2
NS = 16
CH = 128


def _tc_table_kernel(x_ref, w_ref, bvec_ref, ei_ref, et_ref,
                     tab_ref, gidx_ref, dst_out_ref, *, plane_rows, spp,
                     bias_plane, n_edges, n_nodes, dummy, n_rel):
    acc = jnp.dot(x_ref[...], w_ref[...], preferred_element_type=jnp.float32)

    @pl.when(pl.program_id(0) == bias_plane)
    def _():
        tab_ref[0] = acc + bvec_ref[...]

    @pl.when(pl.program_id(0) != bias_plane)
    def _():
        tab_ref[0] = acc

    @pl.when(pl.program_id(0) == 0)
    def _():
        shp = gidx_ref.shape
        eid = (lax.broadcasted_iota(jnp.int32, shp, 0) * 128
               + lax.broadcasted_iota(jnp.int32, shp, 1))
        node = eid - n_edges
        et = jnp.where(eid < n_edges, et_ref[...], n_rel)
        src = jnp.where(eid < n_edges, ei_ref[:, 0, :], node)
        is_pad = eid >= n_edges + n_nodes
        gidx = (et // spp) * plane_rows + src * spp + (et % spp)
        gidx_ref[...] = jnp.where(is_pad, 0, gidx)
        dst = jnp.where(eid < n_edges, ei_ref[:, 1, :], node)
        dst_out_ref[...] = jnp.where(is_pad, dummy, dst)


def _tc_combine_kernel(p_ref, o_ref):
    o_ref[...] = p_ref[0] + p_ref[1]


def _sc_scatter_body(table_hbm, gidx_hbm, dst_hbm, zrows_hbm, out_hbm,
                     gidx_v, dst_v, bufs, vout, acc_sh,
                     gsems, ssems, *, nchunk, rows_per_tile, nbuf):
    c = lax.axis_index("c")
    s = lax.axis_index("s")
    wid = s * NC + c

    pltpu.sync_copy(gidx_hbm.at[wid], gidx_v)
    pltpu.sync_copy(dst_hbm.at[wid], dst_v)

    pltpu.sync_copy(zrows_hbm, vout)
    pltpu.sync_copy(vout, acc_sh.at[pl.ds(s * rows_per_tile, rows_per_tile)])
    plsc.subcore_barrier()

    def start_gather(j, b):
        pltpu.async_copy(table_hbm.at[gidx_v.at[j]], bufs.at[b], gsems.at[b])

    def wait_gather(b):
        pltpu.make_async_copy(table_hbm.at[gidx_v.at[0]], bufs.at[b],
                              gsems.at[b]).wait()

    def start_scatter(j, b):
        pltpu.async_copy(bufs.at[b], acc_sh.at[dst_v.at[j]], ssems.at[b],
                         add=True)

    def wait_scatter(b):
        pltpu.make_async_copy(bufs.at[b], acc_sh.at[dst_v.at[0]],
                              ssems.at[b]).wait()

    start_gather(0, 0)
    start_gather(1, 1)
    wait_gather(0)
    start_scatter(0, 0)
    start_gather(2, 2)

    def body(i, carry):
        for t in range(3):
            j = 3 * i + 1 + t
            b = (1 + t) % 3
            wait_gather(b)
            wait_scatter((b + 2) % 3)
            start_scatter(j, b)

            @pl.when(j + 2 < nchunk)
            def _(j=j, b=b):
                start_gather(j + 2, (b + 2) % 3)
        return carry

    lax.fori_loop(0, (nchunk - 1) // 3, body, 0)
    wait_scatter((nchunk - 1) % 3)
    plsc.subcore_barrier()

    pltpu.sync_copy(acc_sh.at[pl.ds(s * rows_per_tile, rows_per_tile)], vout)
    pltpu.sync_copy(vout, out_hbm.at[c, pl.ds(s * rows_per_tile, rows_per_tile)])


def kernel(x, edge_index, edge_type, W, loop_weight, bias):
    n, d_in = x.shape
    e = edge_type.shape[0]
    r = W.shape[0]
    d_out = W.shape[2]
    f32 = jnp.float32

    slot = 8
    spp = 128 // slot
    r_pad = ((r + 1 + spp - 1) // spp) * spp
    n_acc = ((n + 1 + 8 * NS - 1) // (8 * NS)) * (8 * NS)
    rows_per_tile = n_acc // NS
    dummy = n
    nw = NC * NS
    nbuf = 3
    e_full = e + n
    nchunk = -(-e_full // (nw * CH))
    while (nchunk - 1) % 3:
        nchunk += 1
    e_pad = nw * nchunk * CH
    ep_rows = e_pad // 128

    w_full = jnp.concatenate([W, loop_weight[None]], axis=0)
    w_pad = jnp.zeros((r_pad, d_in, slot), f32).at[:r + 1, :, :d_out].set(w_full)
    w_cat = w_pad.transpose(1, 0, 2).reshape(d_in, r_pad * slot)
    bias_plane = r // spp
    bias_slot = r % spp
    bvec = jnp.zeros((1, 128), f32).at[0, bias_slot * slot:
                                       bias_slot * slot + d_out].set(
        bias.astype(f32))

    e_rows = e // 128
    ei3 = jnp.pad(edge_index.reshape(2, e_rows, 128).transpose(1, 0, 2),
                  ((0, ep_rows - e_rows), (0, 0), (0, 0)))
    et_r = jnp.pad(edge_type.reshape(e_rows, 128),
                   ((0, ep_rows - e_rows), (0, 0)))

    zrows = jnp.zeros((rows_per_tile, slot), f32)

    planes = r_pad * slot // 128
    plane_rows = n * 128 // slot
    table, gidx, dst_syn = pl.pallas_call(
        functools.partial(_tc_table_kernel, plane_rows=plane_rows, spp=spp,
                          bias_plane=bias_plane, n_edges=e, n_nodes=n,
                          dummy=dummy, n_rel=r),
        grid=(planes,),
        in_specs=[
            pl.BlockSpec((n, d_in), lambda g: (0, 0)),
            pl.BlockSpec((d_in, 128), lambda g: (0, g)),
            pl.BlockSpec((1, 128), lambda g: (0, 0)),
            pl.BlockSpec((ep_rows, 2, 128), lambda g: (0, 0, 0)),
            pl.BlockSpec((ep_rows, 128), lambda g: (0, 0)),
        ],
        out_specs=[
            pl.BlockSpec((1, n, 128), lambda g: (g, 0, 0)),
            pl.BlockSpec((ep_rows, 128), lambda g: (0, 0)),
            pl.BlockSpec((ep_rows, 128), lambda g: (0, 0)),
        ],
        out_shape=[
            jax.ShapeDtypeStruct((planes, n, 128), f32),
            jax.ShapeDtypeStruct((ep_rows, 128), jnp.int32),
            jax.ShapeDtypeStruct((ep_rows, 128), jnp.int32),
        ],
    )(x, w_cat, bvec, ei3, et_r)

    mesh = plsc.VectorSubcoreMesh(core_axis_name="c", subcore_axis_name="s",
                                  num_cores=NC, num_subcores=NS)
    sc = pl.kernel(
        functools.partial(_sc_scatter_body, nchunk=nchunk,
                          rows_per_tile=rows_per_tile, nbuf=nbuf),
        out_type=jax.ShapeDtypeStruct((NC, n_acc, slot), f32),
        mesh=mesh,
        compiler_params=pltpu.CompilerParams(use_tc_tiling_on_sc=False),
        scratch_types=[
            pltpu.VMEM((nchunk, CH), jnp.int32),
            pltpu.VMEM((nchunk, CH), jnp.int32),
            pltpu.VMEM((nbuf, CH, slot), f32),
            pltpu.VMEM((rows_per_tile, slot), f32),
            pltpu.VMEM_SHARED((n_acc, slot), f32),
            pltpu.SemaphoreType.DMA((nbuf,)),
            pltpu.SemaphoreType.DMA((nbuf,)),
        ],
    )
    partials = sc(table.reshape(n * r_pad, slot),
                  gidx.reshape(nw, nchunk, CH),
                  dst_syn.reshape(nw, nchunk, CH), zrows)

    flat_rows = n_acc * slot // 128
    out_flat = pl.pallas_call(
        _tc_combine_kernel,
        grid=(1,),
        in_specs=[
            pl.BlockSpec((NC, flat_rows, 128), lambda g: (0, 0, 0)),
        ],
        out_specs=pl.BlockSpec((flat_rows, 128), lambda g: (0, 0)),
        out_shape=jax.ShapeDtypeStruct((flat_rows, 128), f32),
    )(partials.reshape(NC, flat_rows, 128))

    return out_flat.reshape(n_acc, slot)[:n, :d_out]


# speedup vs baseline: 2.6155x; 1.0289x over previous
import functools

import jax
import jax.numpy as jnp
from jax import lax
from jax.experimental import pallas as pl
from jax.experimental.pallas import tpu as pltpu
from jax.experimental.pallas import tpu_sc as plsc

NC = 2
NS = 16
CH = 128


def _tc_table_kernel(x_ref, w_ref, bvec_ref, ei_ref, et_ref,
                     tab_ref, gidx_ref, dst_out_ref, *, plane_rows, spp,
                     bias_plane, n_edges, n_nodes, dummy, n_rel):
    acc = jnp.dot(x_ref[...], w_ref[...], preferred_element_type=jnp.float32)

    @pl.when(pl.program_id(0) == bias_plane)
    def _():
        tab_ref[0] = acc + bvec_ref[...]

    @pl.when(pl.program_id(0) != bias_plane)
    def _():
        tab_ref[0] = acc

    shp = gidx_ref.shape
    eid = (lax.broadcasted_iota(jnp.int32, shp, 0) * 128
           + lax.broadcasted_iota(jnp.int32, shp, 1)
           + pl.program_id(0) * (shp[0] * 128))
    node = eid - n_edges
    et = jnp.where(eid < n_edges, et_ref[...], n_rel)
    src = jnp.where(eid < n_edges, ei_ref[:, 0, :], node)
    is_pad = eid >= n_edges + n_nodes
    gidx = (et // spp) * plane_rows + src * spp + (et % spp)
    gidx_ref[...] = jnp.where(is_pad, 0, gidx)
    dst = jnp.where(eid < n_edges, ei_ref[:, 1, :], node)
    dst_out_ref[...] = jnp.where(is_pad, dummy, dst)


def _tc_combine_kernel(p_ref, o_ref):
    o_ref[...] = p_ref[0] + p_ref[1]


def _sc_scatter_body(table_hbm, gidx_hbm, dst_hbm, zrows_hbm, out_hbm,
                     gidx_v, dst_v, bufs, vout, acc_sh,
                     gsems, ssems, *, nchunk, rows_per_tile, nbuf):
    c = lax.axis_index("c")
    s = lax.axis_index("s")
    wid = s * NC + c

    pltpu.sync_copy(gidx_hbm.at[wid], gidx_v)
    pltpu.sync_copy(dst_hbm.at[wid], dst_v)

    pltpu.sync_copy(zrows_hbm, vout)
    pltpu.sync_copy(vout, acc_sh.at[pl.ds(s * rows_per_tile, rows_per_tile)])
    plsc.subcore_barrier()

    def start_gather(j, b):
        pltpu.async_copy(table_hbm.at[gidx_v.at[j]], bufs.at[b], gsems.at[b])

    def wait_gather(b):
        pltpu.make_async_copy(table_hbm.at[gidx_v.at[0]], bufs.at[b],
                              gsems.at[b]).wait()

    def start_scatter(j, b):
        pltpu.async_copy(bufs.at[b], acc_sh.at[dst_v.at[j]], ssems.at[b],
                         add=True)

    def wait_scatter(b):
        pltpu.make_async_copy(bufs.at[b], acc_sh.at[dst_v.at[0]],
                              ssems.at[b]).wait()

    start_gather(0, 0)
    start_gather(1, 1)
    wait_gather(0)
    start_scatter(0, 0)
    start_gather(2, 2)

    def body(i, carry):
        for t in range(3):
            j = 3 * i + 1 + t
            b = (1 + t) % 3
            wait_gather(b)
            wait_scatter((b + 2) % 3)
            start_scatter(j, b)

            @pl.when(j + 2 < nchunk)
            def _(j=j, b=b):
                start_gather(j + 2, (b + 2) % 3)
        return carry

    lax.fori_loop(0, (nchunk - 1) // 3, body, 0)
    wait_scatter((nchunk - 1) % 3)
    plsc.subcore_barrier()

    pltpu.sync_copy(acc_sh.at[pl.ds(s * rows_per_tile, rows_per_tile)], vout)
    pltpu.sync_copy(vout, out_hbm.at[c, pl.ds(s * rows_per_tile, rows_per_tile)])


def kernel(x, edge_index, edge_type, W, loop_weight, bias):
    n, d_in = x.shape
    e = edge_type.shape[0]
    r = W.shape[0]
    d_out = W.shape[2]
    f32 = jnp.float32

    slot = 8
    spp = 128 // slot
    r_pad = ((r + 1 + spp - 1) // spp) * spp
    n_acc = ((n + 1 + 8 * NS - 1) // (8 * NS)) * (8 * NS)
    rows_per_tile = n_acc // NS
    dummy = n
    nw = NC * NS
    nbuf = 3
    e_full = e + n
    nchunk = -(-e_full // (nw * CH))
    while (nchunk - 1) % 3:
        nchunk += 1
    e_pad = nw * nchunk * CH
    ep_rows = e_pad // 128

    w_full = jnp.concatenate([W, loop_weight[None]], axis=0)
    w_pad = jnp.zeros((r_pad, d_in, slot), f32).at[:r + 1, :, :d_out].set(w_full)
    w_cat = w_pad.transpose(1, 0, 2).reshape(d_in, r_pad * slot)
    bias_plane = r // spp
    bias_slot = r % spp
    bvec = jnp.zeros((1, 128), f32).at[0, bias_slot * slot:
                                       bias_slot * slot + d_out].set(
        bias.astype(f32))

    e_rows = e // 128
    ei3 = jnp.pad(edge_index.reshape(2, e_rows, 128).transpose(1, 0, 2),
                  ((0, ep_rows - e_rows), (0, 0), (0, 0)))
    et_r = jnp.pad(edge_type.reshape(e_rows, 128),
                   ((0, ep_rows - e_rows), (0, 0)))

    zrows = jnp.zeros((rows_per_tile, slot), f32)

    planes = r_pad * slot // 128
    plane_rows = n * 128 // slot
    table, gidx, dst_syn = pl.pallas_call(
        functools.partial(_tc_table_kernel, plane_rows=plane_rows, spp=spp,
                          bias_plane=bias_plane, n_edges=e, n_nodes=n,
                          dummy=dummy, n_rel=r),
        grid=(planes,),
        in_specs=[
            pl.BlockSpec((n, d_in), lambda g: (0, 0)),
            pl.BlockSpec((d_in, 128), lambda g: (0, g)),
            pl.BlockSpec((1, 128), lambda g: (0, 0)),
            pl.BlockSpec((ep_rows // planes, 2, 128), lambda g: (g, 0, 0)),
            pl.BlockSpec((ep_rows // planes, 128), lambda g: (g, 0)),
        ],
        out_specs=[
            pl.BlockSpec((1, n, 128), lambda g: (g, 0, 0)),
            pl.BlockSpec((ep_rows // planes, 128), lambda g: (g, 0)),
            pl.BlockSpec((ep_rows // planes, 128), lambda g: (g, 0)),
        ],
        out_shape=[
            jax.ShapeDtypeStruct((planes, n, 128), f32),
            jax.ShapeDtypeStruct((ep_rows, 128), jnp.int32),
            jax.ShapeDtypeStruct((ep_rows, 128), jnp.int32),
        ],
    )(x, w_cat, bvec, ei3, et_r)

    mesh = plsc.VectorSubcoreMesh(core_axis_name="c", subcore_axis_name="s",
                                  num_cores=NC, num_subcores=NS)
    sc = pl.kernel(
        functools.partial(_sc_scatter_body, nchunk=nchunk,
                          rows_per_tile=rows_per_tile, nbuf=nbuf),
        out_type=jax.ShapeDtypeStruct((NC, n_acc, slot), f32),
        mesh=mesh,
        compiler_params=pltpu.CompilerParams(use_tc_tiling_on_sc=False),
        scratch_types=[
            pltpu.VMEM((nchunk, CH), jnp.int32),
            pltpu.VMEM((nchunk, CH), jnp.int32),
            pltpu.VMEM((nbuf, CH, slot), f32),
            pltpu.VMEM((rows_per_tile, slot), f32),
            pltpu.VMEM_SHARED((n_acc, slot), f32),
            pltpu.SemaphoreType.DMA((nbuf,)),
            pltpu.SemaphoreType.DMA((nbuf,)),
        ],
    )
    partials = sc(table.reshape(n * r_pad, slot),
                  gidx.reshape(nw, nchunk, CH),
                  dst_syn.reshape(nw, nchunk, CH), zrows)

    flat_rows = n_acc * slot // 128
    out_flat = pl.pallas_call(
        _tc_combine_kernel,
        grid=(1,),
        in_specs=[
            pl.BlockSpec((NC, flat_rows, 128), lambda g: (0, 0, 0)),
        ],
        out_specs=pl.BlockSpec((flat_rows, 128), lambda g: (0, 0)),
        out_shape=jax.ShapeDtypeStruct((flat_rows, 128), f32),
    )(partials.reshape(NC, flat_rows, 128))

    return out_flat.reshape(n_acc, slot)[:n, :d_out]


# speedup vs baseline: 2.7187x; 1.0394x over previous
import functools

import jax
import jax.numpy as jnp
from jax import lax
from jax.experimental import pallas as pl
from jax.experimental.pallas import tpu as pltpu
from jax.experimental.pallas import tpu_sc as plsc

NC = 2
NS = 16
CH = 128


def _tc_table_kernel(x_ref, w_ref, bvec_ref, ei_ref, et_ref,
                     tab_ref, gidx_ref, dst_out_ref, *, plane_rows, spp,
                     bias_plane, n_edges, n_nodes, dummy, n_rel):
    acc = jnp.dot(x_ref[...], w_ref[...], preferred_element_type=jnp.float32)

    @pl.when(pl.program_id(0) == bias_plane)
    def _():
        tab_ref[0] = acc + bvec_ref[...]

    @pl.when(pl.program_id(0) != bias_plane)
    def _():
        tab_ref[0] = acc

    shp = gidx_ref.shape
    eid = (lax.broadcasted_iota(jnp.int32, shp, 0) * 128
           + lax.broadcasted_iota(jnp.int32, shp, 1)
           + pl.program_id(0) * (shp[0] * 128))
    node = eid - n_edges
    et = jnp.where(eid < n_edges, et_ref[...], n_rel)
    src = jnp.where(eid < n_edges, ei_ref[:, 0, :], node)
    is_pad = eid >= n_edges + n_nodes
    gidx = (et // spp) * plane_rows + src * spp + (et % spp)
    gidx_ref[...] = jnp.where(is_pad, 0, gidx)
    dst = jnp.where(eid < n_edges, ei_ref[:, 1, :], node)
    dst_out_ref[...] = jnp.where(is_pad, dummy, dst)


def _tc_combine_kernel(p_ref, o_ref):
    o_ref[...] = p_ref[0] + p_ref[1]


def _sc_scatter_body(table_hbm, gidx_hbm, dst_hbm, zrows_hbm, out_hbm,
                     gidx_v, dst_v, bufs, vout, acc_sh,
                     gsems, ssems, *, nchunk, rows_per_tile, nbuf):
    c = lax.axis_index("c")
    s = lax.axis_index("s")
    wid = s * NC + c

    pltpu.sync_copy(gidx_hbm.at[wid], gidx_v)
    pltpu.sync_copy(dst_hbm.at[wid], dst_v)

    pltpu.sync_copy(zrows_hbm, vout)
    pltpu.sync_copy(vout, acc_sh.at[pl.ds(s * rows_per_tile, rows_per_tile)])
    plsc.subcore_barrier()

    def start_gather(j, b):
        pltpu.async_copy(table_hbm.at[gidx_v.at[j]], bufs.at[b], gsems.at[b])

    def wait_gather(b):
        pltpu.make_async_copy(table_hbm.at[gidx_v.at[0]], bufs.at[b],
                              gsems.at[b]).wait()

    def start_scatter(j, b):
        pltpu.async_copy(bufs.at[b], acc_sh.at[dst_v.at[j]], ssems.at[b],
                         add=True)

    def wait_scatter(b):
        pltpu.make_async_copy(bufs.at[b], acc_sh.at[dst_v.at[0]],
                              ssems.at[b]).wait()

    start_gather(0, 0)
    start_gather(1, 1)
    wait_gather(0)
    start_scatter(0, 0)
    start_gather(2, 2)

    def body(i, carry):
        for t in range(3):
            j = 3 * i + 1 + t
            b = (1 + t) % 3
            wait_gather(b)
            wait_scatter((b + 2) % 3)
            start_scatter(j, b)

            @pl.when(j + 2 < nchunk)
            def _(j=j, b=b):
                start_gather(j + 2, (b + 2) % 3)
        return carry

    lax.fori_loop(0, (nchunk - 1) // 3, body, 0)
    wait_scatter((nchunk - 1) % 3)
    plsc.subcore_barrier()

    pltpu.sync_copy(acc_sh.at[pl.ds(s * rows_per_tile, rows_per_tile)], vout)
    pltpu.sync_copy(vout, out_hbm.at[c, pl.ds(s * rows_per_tile, rows_per_tile)])


def kernel(x, edge_index, edge_type, W, loop_weight, bias):
    n, d_in = x.shape
    e = edge_type.shape[0]
    r = W.shape[0]
    d_out = W.shape[2]
    f32 = jnp.float32

    slot = 8
    spp = 128 // slot
    r_pad = ((r + 1 + spp - 1) // spp) * spp
    n_acc = ((n + 1 + 8 * NS - 1) // (8 * NS)) * (8 * NS)
    rows_per_tile = n_acc // NS
    dummy = n
    nw = NC * NS
    nbuf = 3
    e_full = e + n
    nchunk = -(-e_full // (nw * CH))
    while (nchunk - 1) % 3:
        nchunk += 1
    e_pad = nw * nchunk * CH
    ep_rows = e_pad // 128

    w_full = jnp.concatenate([W, loop_weight[None]], axis=0)
    w_pad = jnp.zeros((r_pad, d_in, slot), f32).at[:r + 1, :, :d_out].set(w_full)
    w_cat = w_pad.transpose(1, 0, 2).reshape(d_in, r_pad * slot)
    bias_plane = r // spp
    bias_slot = r % spp
    bvec = jnp.zeros((1, 128), f32).at[0, bias_slot * slot:
                                       bias_slot * slot + d_out].set(
        bias.astype(f32))

    e_rows = e // 128
    ei3 = edge_index.reshape(2, e_rows, 128).transpose(1, 0, 2)
    et_r = edge_type.reshape(e_rows, 128)

    zrows = jnp.zeros((rows_per_tile, slot), f32)

    planes = r_pad * slot // 128
    plane_rows = n * 128 // slot
    table, gidx, dst_syn = pl.pallas_call(
        functools.partial(_tc_table_kernel, plane_rows=plane_rows, spp=spp,
                          bias_plane=bias_plane, n_edges=e, n_nodes=n,
                          dummy=dummy, n_rel=r),
        grid=(planes,),
        in_specs=[
            pl.BlockSpec((n, d_in), lambda g: (0, 0)),
            pl.BlockSpec((d_in, 128), lambda g: (0, g)),
            pl.BlockSpec((1, 128), lambda g: (0, 0)),
            pl.BlockSpec((ep_rows // planes, 2, 128), lambda g: (g, 0, 0)),
            pl.BlockSpec((ep_rows // planes, 128), lambda g: (g, 0)),
        ],
        out_specs=[
            pl.BlockSpec((1, n, 128), lambda g: (g, 0, 0)),
            pl.BlockSpec((ep_rows // planes, 128), lambda g: (g, 0)),
            pl.BlockSpec((ep_rows // planes, 128), lambda g: (g, 0)),
        ],
        out_shape=[
            jax.ShapeDtypeStruct((planes, n, 128), f32),
            jax.ShapeDtypeStruct((ep_rows, 128), jnp.int32),
            jax.ShapeDtypeStruct((ep_rows, 128), jnp.int32),
        ],
    )(x, w_cat, bvec, ei3, et_r)

    mesh = plsc.VectorSubcoreMesh(core_axis_name="c", subcore_axis_name="s",
                                  num_cores=NC, num_subcores=NS)
    sc = pl.kernel(
        functools.partial(_sc_scatter_body, nchunk=nchunk,
                          rows_per_tile=rows_per_tile, nbuf=nbuf),
        out_type=jax.ShapeDtypeStruct((NC, n_acc, slot), f32),
        mesh=mesh,
        compiler_params=pltpu.CompilerParams(use_tc_tiling_on_sc=False),
        scratch_types=[
            pltpu.VMEM((nchunk, CH), jnp.int32),
            pltpu.VMEM((nchunk, CH), jnp.int32),
            pltpu.VMEM((nbuf, CH, slot), f32),
            pltpu.VMEM((rows_per_tile, slot), f32),
            pltpu.VMEM_SHARED((n_acc, slot), f32),
            pltpu.SemaphoreType.DMA((nbuf,)),
            pltpu.SemaphoreType.DMA((nbuf,)),
        ],
    )
    partials = sc(table.reshape(n * r_pad, slot),
                  gidx.reshape(nw, nchunk, CH),
                  dst_syn.reshape(nw, nchunk, CH), zrows)

    flat_rows = n_acc * slot // 128
    out_flat = pl.pallas_call(
        _tc_combine_kernel,
        grid=(1,),
        in_specs=[
            pl.BlockSpec((NC, flat_rows, 128), lambda g: (0, 0, 0)),
        ],
        out_specs=pl.BlockSpec((flat_rows, 128), lambda g: (0, 0)),
        out_shape=jax.ShapeDtypeStruct((flat_rows, 128), f32),
    )(partials.reshape(NC, flat_rows, 128))

    return out_flat.reshape(n_acc, slot)[:n, :d_out]
